# baseline (speedup 1.0000x reference)
import functools

import jax
import jax.numpy as jnp
from jax import lax
from jax.experimental import pallas as pl
from jax.experimental.pallas import tpu as pltpu
from jax.experimental.pallas import tpu_sc as plsc

BATCH = 1024
SEQ = 200
DIM = 300
TOTAL = BATCH * SEQ
LANES = 128
TAIL = DIM - 2 * LANES
WIDTH = 3 * LANES

CHUNK = 128
NGROUP = CHUNK // 8
NBUF = 2


@functools.lru_cache(maxsize=None)
def _build(total):
    info = plsc.get_sparse_core_info()
    nw = info.num_cores * info.num_subcores
    b_per_w = total // nw
    n_chunks = b_per_w // CHUNK
    assert n_chunks % NBUF == 0
    mesh = plsc.VectorSubcoreMesh(core_axis_name="c", subcore_axis_name="s")

    @functools.partial(
        pl.kernel,
        mesh=mesh,
        out_type=jax.ShapeDtypeStruct((total // 8, 8, WIDTH), jnp.float32),
        scratch_types=[
            pltpu.VMEM((b_per_w,), jnp.int32),
            *[pltpu.VMEM((CHUNK, WIDTH), jnp.float32) for _ in range(NBUF)],
            *[pltpu.SemaphoreType.DMA for _ in range(2 * NBUF)],
        ],
    )
    def gather_kernel(idx_hbm, table_hbm, aux_hbm, out_hbm, idx_all,
                      *bufs_sems):
        bufs = bufs_sems[:NBUF]
        gsems = bufs_sems[NBUF:2 * NBUF]
        osems = bufs_sems[2 * NBUF:]
        wid = lax.axis_index("s") * info.num_cores + lax.axis_index("c")
        wbase = wid * b_per_w
        wg = wbase // 8

        pltpu.sync_copy(idx_hbm.at[pl.ds(wbase, b_per_w)], idx_all)

        def start_gathers(i, b):
            sl = idx_all.at[pl.ds(i * CHUNK, CHUNK)]
            m = bufs[b]
            pltpu.async_copy(
                table_hbm.at[plsc.Indices(sl), pl.ds(0, LANES)],
                m.at[:, pl.ds(0, LANES)], gsems[b])
            pltpu.async_copy(
                table_hbm.at[plsc.Indices(sl), pl.ds(LANES, LANES)],
                m.at[:, pl.ds(LANES, LANES)], gsems[b])
            pltpu.async_copy(
                aux_hbm.at[plsc.Indices(sl)],
                m.at[:, pl.ds(2 * LANES, LANES)], gsems[b])

        def wait_gathers(i, b):
            sl = idx_all.at[pl.ds(i * CHUNK, CHUNK)]
            m = bufs[b]
            pltpu.make_async_copy(
                table_hbm.at[plsc.Indices(sl), pl.ds(0, LANES)],
                m.at[:, pl.ds(0, LANES)], gsems[b]).wait()
            pltpu.make_async_copy(
                table_hbm.at[plsc.Indices(sl), pl.ds(LANES, LANES)],
                m.at[:, pl.ds(LANES, LANES)], gsems[b]).wait()
            pltpu.make_async_copy(
                aux_hbm.at[plsc.Indices(sl)],
                m.at[:, pl.ds(2 * LANES, LANES)], gsems[b]).wait()

        def issue_tile_writes(i, b):
            def group_body(g, carry):
                pltpu.async_copy(
                    bufs[b].at[pl.ds(8 * g, 8)],
                    out_hbm.at[wg + i * NGROUP + g], osems[b])
                return carry

            lax.fori_loop(0, NGROUP, group_body, 0)

        def wait_tile_writes(i, b):
            def group_body(g, carry):
                pltpu.make_async_copy(
                    bufs[b].at[pl.ds(8 * g, 8)],
                    out_hbm.at[wg + i * NGROUP + g], osems[b]).wait()
                return carry

            lax.fori_loop(0, NGROUP, group_body, 0)

        for j in range(NBUF - 1):
            start_gathers(j, j)

        def outer(g, carry):
            for b in range(NBUF):
                i = NBUF * g + b
                wait_gathers(i, b)
                issue_tile_writes(i, b)
                nb = (b + NBUF - 1) % NBUF

                @pl.when(i + NBUF - 1 < n_chunks)
                def _():
                    @pl.when(i >= 1)
                    def _():
                        wait_tile_writes(i - 1, nb)

                    start_gathers(i + NBUF - 1, nb)
            return carry

        lax.fori_loop(0, n_chunks // NBUF, outer, 0)

        for j in range(NBUF):
            i = n_chunks - NBUF + j
            wait_tile_writes(i, i % NBUF)

    return gather_kernel


def kernel(x, word_vectors):
    idx = x.reshape(-1).astype(jnp.int32)
    aux = jnp.pad(word_vectors[:, 2 * LANES:], ((0, 0), (0, LANES - TAIL)))
    out = _build(TOTAL)(idx, word_vectors, aux)
    return out[:, :, :DIM].reshape(BATCH, SEQ, DIM)


# speedup vs baseline: 1.0037x; 1.0037x over previous
import functools

import jax
import jax.numpy as jnp
from jax import lax
from jax.experimental import pallas as pl
from jax.experimental.pallas import tpu as pltpu
from jax.experimental.pallas import tpu_sc as plsc

BATCH = 1024
SEQ = 200
DIM = 300
TOTAL = BATCH * SEQ
LANES = 128
TAIL = DIM - 2 * LANES
WIDTH = 3 * LANES

CHUNK = 64
NGROUP = CHUNK // 8
NBUF = 4


@functools.lru_cache(maxsize=None)
def _build(total):
    info = plsc.get_sparse_core_info()
    nw = info.num_cores * info.num_subcores
    b_per_w = total // nw
    n_chunks = b_per_w // CHUNK
    assert n_chunks % NBUF == 0
    mesh = plsc.VectorSubcoreMesh(core_axis_name="c", subcore_axis_name="s")

    @functools.partial(
        pl.kernel,
        mesh=mesh,
        out_type=jax.ShapeDtypeStruct((total // 8, 8, WIDTH), jnp.float32),
        scratch_types=[
            pltpu.VMEM((b_per_w,), jnp.int32),
            *[pltpu.VMEM((CHUNK, WIDTH), jnp.float32) for _ in range(NBUF)],
            *[pltpu.SemaphoreType.DMA for _ in range(2 * NBUF)],
        ],
    )
    def gather_kernel(idx_hbm, table_hbm, aux_hbm, out_hbm, idx_all,
                      *bufs_sems):
        bufs = bufs_sems[:NBUF]
        gsems = bufs_sems[NBUF:2 * NBUF]
        osems = bufs_sems[2 * NBUF:]
        wid = lax.axis_index("s") * info.num_cores + lax.axis_index("c")
        wbase = wid * b_per_w
        wg = wbase // 8

        pltpu.sync_copy(idx_hbm.at[pl.ds(wbase, b_per_w)], idx_all)

        def start_gathers(i, b):
            sl = idx_all.at[pl.ds(i * CHUNK, CHUNK)]
            m = bufs[b]
            pltpu.async_copy(
                table_hbm.at[plsc.Indices(sl), pl.ds(0, LANES)],
                m.at[:, pl.ds(0, LANES)], gsems[b])
            pltpu.async_copy(
                table_hbm.at[plsc.Indices(sl), pl.ds(LANES, LANES)],
                m.at[:, pl.ds(LANES, LANES)], gsems[b])
            pltpu.async_copy(
                aux_hbm.at[plsc.Indices(sl)],
                m.at[:, pl.ds(2 * LANES, LANES)], gsems[b])

        def wait_gathers(i, b):
            sl = idx_all.at[pl.ds(i * CHUNK, CHUNK)]
            m = bufs[b]
            pltpu.make_async_copy(
                table_hbm.at[plsc.Indices(sl), pl.ds(0, LANES)],
                m.at[:, pl.ds(0, LANES)], gsems[b]).wait()
            pltpu.make_async_copy(
                table_hbm.at[plsc.Indices(sl), pl.ds(LANES, LANES)],
                m.at[:, pl.ds(LANES, LANES)], gsems[b]).wait()
            pltpu.make_async_copy(
                aux_hbm.at[plsc.Indices(sl)],
                m.at[:, pl.ds(2 * LANES, LANES)], gsems[b]).wait()

        def issue_tile_writes(i, b):
            def group_body(g, carry):
                pltpu.async_copy(
                    bufs[b].at[pl.ds(8 * g, 8)],
                    out_hbm.at[wg + i * NGROUP + g], osems[b])
                return carry

            lax.fori_loop(0, NGROUP, group_body, 0)

        def wait_tile_writes(i, b):
            def group_body(g, carry):
                pltpu.make_async_copy(
                    bufs[b].at[pl.ds(8 * g, 8)],
                    out_hbm.at[wg + i * NGROUP + g], osems[b]).wait()
                return carry

            lax.fori_loop(0, NGROUP, group_body, 0)

        for j in range(NBUF - 1):
            start_gathers(j, j)

        def outer(g, carry):
            for b in range(NBUF):
                i = NBUF * g + b
                wait_gathers(i, b)
                issue_tile_writes(i, b)
                nb = (b + NBUF - 1) % NBUF

                @pl.when(i + NBUF - 1 < n_chunks)
                def _():
                    @pl.when(i >= 1)
                    def _():
                        wait_tile_writes(i - 1, nb)

                    start_gathers(i + NBUF - 1, nb)
            return carry

        lax.fori_loop(0, n_chunks // NBUF, outer, 0)

        for j in range(NBUF):
            i = n_chunks - NBUF + j
            wait_tile_writes(i, i % NBUF)

    return gather_kernel


def kernel(x, word_vectors):
    idx = x.reshape(-1).astype(jnp.int32)
    aux = jnp.pad(word_vectors[:, 2 * LANES:], ((0, 0), (0, LANES - TAIL)))
    out = _build(TOTAL)(idx, word_vectors, aux)
    return out[:, :, :DIM].reshape(BATCH, SEQ, DIM)
